# baseline (speedup 1.0000x reference)
import functools

import jax
import jax.numpy as jnp
from jax import lax
from jax.experimental import pallas as pl
from jax.experimental.pallas import tpu as pltpu
from jax.experimental.pallas import tpu_sc as plsc


def kernel(x, weight):
    batch, seq = x.shape
    dim = weight.shape[1]
    info = plsc.get_sparse_core_info()
    nw = info.num_cores * info.num_subcores
    rows_per_w = seq // nw

    mesh = plsc.VectorSubcoreMesh(core_axis_name="c", subcore_axis_name="s")

    @functools.partial(
        pl.kernel,
        mesh=mesh,
        out_type=jax.ShapeDtypeStruct((batch, seq, dim), weight.dtype),
        scratch_types=[
            pltpu.SemaphoreType.DMA,
        ],
    )
    def _sc_bcast(w_hbm, o_hbm, sem):
        wid = lax.axis_index("s") * info.num_cores + lax.axis_index("c")
        base = wid * rows_per_w

        copies = [
            pltpu.make_async_copy(
                w_hbm.at[pl.ds(base, rows_per_w), :],
                o_hbm.at[b, pl.ds(base, rows_per_w), :],
                sem,
            )
            for b in range(batch)
        ]
        for c in copies:
            c.start()
        for c in copies:
            c.wait()

    return _sc_bcast(weight)


# speedup vs baseline: 52.8051x; 52.8051x over previous
import functools

import jax
import jax.numpy as jnp
from jax import lax
from jax.experimental import pallas as pl
from jax.experimental.pallas import tpu as pltpu
from jax.experimental.pallas import tpu_sc as plsc

_CHUNK = 32


def kernel(x, weight):
    batch, seq = x.shape
    dim = weight.shape[1]
    info = plsc.get_sparse_core_info()
    nw = info.num_cores * info.num_subcores
    rows_per_w = seq // nw
    nchunk = rows_per_w // _CHUNK

    mesh = plsc.VectorSubcoreMesh(core_axis_name="c", subcore_axis_name="s")

    @functools.partial(
        pl.kernel,
        mesh=mesh,
        out_type=jax.ShapeDtypeStruct((batch, seq, dim), weight.dtype),
        scratch_types=[
            pltpu.VMEM((2, _CHUNK, dim), weight.dtype),
            pltpu.SemaphoreType.DMA((2,)),
            pltpu.SemaphoreType.DMA((2,)),
        ],
    )
    def _sc_bcast(w_hbm, o_hbm, buf, sem_r, sem_w):
        wid = lax.axis_index("s") * info.num_cores + lax.axis_index("c")
        base = wid * rows_per_w

        def read(i, slot):
            return pltpu.make_async_copy(
                w_hbm.at[pl.ds(base + i * _CHUNK, _CHUNK), :],
                buf.at[slot],
                sem_r.at[slot],
            )

        def write(i, slot, b):
            return pltpu.make_async_copy(
                buf.at[slot],
                o_hbm.at[b, pl.ds(base + i * _CHUNK, _CHUNK), :],
                sem_w.at[slot],
            )

        read(0, 0).start()
        for i in range(nchunk):
            slot = i % 2
            read(i, slot).wait()
            for b in range(batch):
                write(i, slot, b).start()
            if i + 1 < nchunk:
                if i >= 1:
                    for b in range(batch):
                        write(i - 1, 1 - slot, b).wait()
                read(i + 1, 1 - slot).start()
        for i in (nchunk - 2, nchunk - 1):
            for b in range(batch):
                write(i, i % 2, b).wait()

    return _sc_bcast(weight)


# speedup vs baseline: 55.8647x; 1.0579x over previous
import functools

import jax
import jax.numpy as jnp
from jax import lax
from jax.experimental import pallas as pl
from jax.experimental.pallas import tpu as pltpu
from jax.experimental.pallas import tpu_sc as plsc

_CHUNK = 64


def kernel(x, weight):
    batch, seq = x.shape
    dim = weight.shape[1]
    info = plsc.get_sparse_core_info()
    nw = info.num_cores * info.num_subcores
    rows_per_w = seq // nw
    nchunk = rows_per_w // _CHUNK

    mesh = plsc.VectorSubcoreMesh(core_axis_name="c", subcore_axis_name="s")

    @functools.partial(
        pl.kernel,
        mesh=mesh,
        out_type=jax.ShapeDtypeStruct((batch, seq, dim), weight.dtype),
        scratch_types=[
            pltpu.VMEM((_CHUNK, dim), weight.dtype),
            pltpu.SemaphoreType.DMA,
        ],
    )
    def _sc_bcast(w_hbm, o_hbm, buf, sem):
        wid = lax.axis_index("s") * info.num_cores + lax.axis_index("c")
        base = wid * rows_per_w

        def body(i, carry):
            r0 = base + i * _CHUNK
            pltpu.sync_copy(w_hbm.at[pl.ds(r0, _CHUNK), :], buf)
            for b in range(batch):
                pltpu.sync_copy(buf, o_hbm.at[b, pl.ds(r0, _CHUNK), :])
            return carry

        lax.fori_loop(0, nchunk, body, 0)

    return _sc_bcast(weight)
